# baseline (speedup 1.0000x reference)
import functools

import jax
import jax.numpy as jnp
from jax import lax
from jax.experimental import pallas as pl
from jax.experimental.pallas import tpu as pltpu
from jax.experimental.pallas import tpu_sc as plsc

_B = 16384
_D = 64
_NW = 32
_EPW = _B // _NW
_C = 128
_NCH = _EPW // _C
_L = 16
_EBLK_ENT = 8192
_HALF_ENT = -(-1000000 // (2 * _EBLK_ENT)) * _EBLK_ENT


def _tx_body(a1_ref, a2_ref, oa_ref):
    oa_ref[...] = jnp.concatenate([a1_ref[...].T, a2_ref[...].T], axis=1)


def _pair_rows(xt, e_blk):
    n = xt.shape[1]
    nb = -(-n // (2 * e_blk))
    h = nb * e_blk
    last = (n - 1) // e_blk
    lo = pl.BlockSpec((_D, e_blk), lambda i: (0, i))
    hi = pl.BlockSpec(
        (_D, e_blk), lambda i, nb=nb, last=last: (0, jnp.minimum(i + nb, last)))
    return pl.pallas_call(
        _tx_body,
        grid=(nb,),
        in_specs=[lo, hi],
        out_specs=pl.BlockSpec((e_blk, 2 * _D), lambda i: (i, 0)),
        out_shape=jax.ShapeDtypeStruct((h, 2 * _D), jnp.float32),
    )(xt, xt)


def _gather_body(head_hbm, tail_hbm, er_hbm,
                 hr_out, tr_out, hp_v, tp_v, row_b, sem):
    wid = lax.axis_index("s") * 2 + lax.axis_index("c")
    base = wid * _EPW

    pltpu.sync_copy(head_hbm.at[pl.ds(base, _EPW)], hp_v)
    pltpu.sync_copy(tail_hbm.at[pl.ds(base, _EPW)], tp_v)

    def xform(g, _):
        for ref in (hp_v, tp_v):
            i = ref[pl.ds(g * _L, _L)]
            ref[pl.ds(g * _L, _L)] = jnp.where(i >= _HALF_ENT,
                                               i - _HALF_ENT, i)
        return 0

    lax.fori_loop(0, _EPW // _L, xform, 0)

    for ch in range(_NCH):
        for idx_v, dst in ((hp_v, hr_out), (tp_v, tr_out)):
            sl = idx_v.at[pl.ds(ch * _C, _C)]
            pltpu.async_copy(er_hbm.at[sl], row_b, sem).wait()
            pltpu.sync_copy(row_b, dst.at[pl.ds(base + ch * _C, _C)])


def _compute_body(head_hbm, rel_hbm, tail_hbm,
                  ei_hbm, rr_hbm, ri_hbm, hr_st, tr_st, out_hbm,
                  h_iv, r_iv, t_iv, out_v,
                  hr_b, hi_b, tr_b, ti_b, rr_b, ri_b, sem):
    wid = lax.axis_index("s") * 2 + lax.axis_index("c")
    base = wid * _EPW

    pltpu.sync_copy(head_hbm.at[pl.ds(base, _EPW)], h_iv)
    pltpu.sync_copy(rel_hbm.at[pl.ds(base, _EPW)], r_iv)
    pltpu.sync_copy(tail_hbm.at[pl.ds(base, _EPW)], t_iv)

    iota = lax.iota(jnp.int32, _L)

    for ch in range(_NCH):
        rsl = r_iv.at[pl.ds(ch * _C, _C)]
        hsl = h_iv.at[pl.ds(ch * _C, _C)]
        tsl = t_iv.at[pl.ds(ch * _C, _C)]
        cps = [
            pltpu.async_copy(hr_st.at[pl.ds(base + ch * _C, _C)], hr_b, sem),
            pltpu.async_copy(tr_st.at[pl.ds(base + ch * _C, _C)], tr_b, sem),
            pltpu.async_copy(ei_hbm.at[hsl], hi_b, sem),
            pltpu.async_copy(ei_hbm.at[tsl], ti_b, sem),
            pltpu.async_copy(rr_hbm.at[rsl], rr_b, sem),
            pltpu.async_copy(ri_hbm.at[rsl], ri_b, sem),
        ]
        for cp in cps:
            cp.wait()

        def grp_body(g, _, ch=ch):
            rows = g * _L + iota
            hraw = h_iv[pl.ds(ch * _C + g * _L, _L)]
            traw = t_iv[pl.ds(ch * _C + g * _L, _L)]
            hc0 = jnp.where(hraw >= _HALF_ENT, 64, 0)
            tc0 = jnp.where(traw >= _HALF_ENT, 64, 0)

            def dim_body(d, acc):
                hr = plsc.load_gather(hr_b, [rows, hc0 + d])
                tr = plsc.load_gather(tr_b, [rows, tc0 + d])
                dd = jnp.zeros((_L,), jnp.int32) + d
                hi = plsc.load_gather(hi_b, [rows, dd])
                ti = plsc.load_gather(ti_b, [rows, dd])
                rr = plsc.load_gather(rr_b, [rows, dd])
                ri = plsc.load_gather(ri_b, [rows, dd])
                return acc + tr * (hr * rr - hi * ri) + ti * (hi * rr + hr * ri)

            acc = lax.fori_loop(0, _D, dim_body, jnp.zeros((_L,), jnp.float32))
            out_v[pl.ds(ch * _C + g * _L, _L)] = acc
            return 0

        lax.fori_loop(0, _C // _L, grp_body, 0)

    pltpu.sync_copy(out_v, out_hbm.at[pl.ds(base, _EPW)])


@jax.jit
def kernel(head, relation, tail, ent_real, ent_imag, rel_real, rel_imag):
    mesh = plsc.VectorSubcoreMesh(core_axis_name="c", subcore_axis_name="s")
    er2 = _pair_rows(ent_real.T, _EBLK_ENT)

    gather = pl.kernel(
        _gather_body,
        out_type=(jax.ShapeDtypeStruct((_B, 2 * _D), jnp.float32),
                  jax.ShapeDtypeStruct((_B, 2 * _D), jnp.float32)),
        mesh=mesh,
        scratch_types=[
            pltpu.VMEM((_EPW,), jnp.int32),
            pltpu.VMEM((_EPW,), jnp.int32),
            pltpu.VMEM((_C, 2 * _D), jnp.float32),
            pltpu.SemaphoreType.DMA,
        ],
        compiler_params=pltpu.CompilerParams(
            needs_layout_passes=False, use_tc_tiling_on_sc=True),
    )
    hr_st, tr_st = gather(head, tail, er2)

    run = pl.kernel(
        _compute_body,
        out_type=jax.ShapeDtypeStruct((_B,), jnp.float32),
        mesh=mesh,
        scratch_types=[
            pltpu.VMEM((_EPW,), jnp.int32),
            pltpu.VMEM((_EPW,), jnp.int32),
            pltpu.VMEM((_EPW,), jnp.int32),
            pltpu.VMEM((_EPW,), jnp.float32),
            pltpu.VMEM((_C, 2 * _D), jnp.float32),
            pltpu.VMEM((_C, _D), jnp.float32),
            pltpu.VMEM((_C, 2 * _D), jnp.float32),
            pltpu.VMEM((_C, _D), jnp.float32),
            pltpu.VMEM((_C, _D), jnp.float32),
            pltpu.VMEM((_C, _D), jnp.float32),
            pltpu.SemaphoreType.DMA,
        ],
        compiler_params=pltpu.CompilerParams(
            needs_layout_passes=False, use_tc_tiling_on_sc=False),
    )
    return run(head, relation, tail, ent_imag, rel_real, rel_imag,
               hr_st, tr_st)


# speedup vs baseline: 1.5981x; 1.5981x over previous
import functools

import jax
import jax.numpy as jnp
from jax import lax
from jax.experimental import pallas as pl
from jax.experimental.pallas import tpu as pltpu
from jax.experimental.pallas import tpu_sc as plsc

_B = 16384
_D = 64
_NW = 32
_EPW = _B // _NW
_C = 64
_NCH = _EPW // _C
_L = 16
_EBLK_ENT = 8192
_EBLK_REL = 512
_HALF_ENT = -(-1000000 // (2 * _EBLK_ENT)) * _EBLK_ENT
_HALF_REL = -(-1000 // (2 * _EBLK_REL)) * _EBLK_REL


def _tx_body(a1_ref, a2_ref, b1_ref, b2_ref, oa_ref, ob_ref):
    oa_ref[...] = jnp.concatenate([a1_ref[...].T, a2_ref[...].T], axis=1)
    ob_ref[...] = jnp.concatenate([b1_ref[...].T, b2_ref[...].T], axis=1)


def _pair_rows(xt, yt, e_blk):
    n = xt.shape[1]
    nb = -(-n // (2 * e_blk))
    h = nb * e_blk
    last = (n - 1) // e_blk
    lo = pl.BlockSpec((_D, e_blk), lambda i: (0, i))
    hi = pl.BlockSpec(
        (_D, e_blk), lambda i, nb=nb, last=last: (0, jnp.minimum(i + nb, last)))
    outs = pl.pallas_call(
        _tx_body,
        grid=(nb,),
        in_specs=[lo, hi, lo, hi],
        out_specs=[
            pl.BlockSpec((e_blk, 2 * _D), lambda i: (i, 0)),
            pl.BlockSpec((e_blk, 2 * _D), lambda i: (i, 0)),
        ],
        out_shape=[
            jax.ShapeDtypeStruct((h, 2 * _D), jnp.float32),
            jax.ShapeDtypeStruct((h, 2 * _D), jnp.float32),
        ],
    )(xt, xt, yt, yt)
    return outs


def _complex_body(head_hbm, rel_hbm, tail_hbm,
                  er_hbm, ei_hbm, rr_hbm, ri_hbm, out_hbm,
                  h_iv, r_iv, t_iv, hp_v, rp_v, tp_v, out_v,
                  hr_b0, hi_b0, tr_b0, ti_b0, rr_b0, ri_b0,
                  hr_b1, hi_b1, tr_b1, ti_b1, rr_b1, ri_b1, sem0, sem1):
    wid = lax.axis_index("s") * 2 + lax.axis_index("c")
    base = wid * _EPW

    pltpu.sync_copy(head_hbm.at[pl.ds(base, _EPW)], h_iv)
    pltpu.sync_copy(rel_hbm.at[pl.ds(base, _EPW)], r_iv)
    pltpu.sync_copy(tail_hbm.at[pl.ds(base, _EPW)], t_iv)

    iota = lax.iota(jnp.int32, _L)

    def xform(g, _):
        for src, dst, half in ((h_iv, hp_v, _HALF_ENT), (r_iv, rp_v, _HALF_REL),
                               (t_iv, tp_v, _HALF_ENT)):
            i = src[pl.ds(g * _L, _L)]
            hi = jnp.where(i >= half, 1, 0)
            dst[pl.ds(g * _L, _L)] = i - hi * half
            src[pl.ds(g * _L, _L)] = hi << 6
        return 0

    lax.fori_loop(0, _EPW // _L, xform, 0)

    bufs = [(hr_b0, hi_b0, tr_b0, ti_b0, rr_b0, ri_b0),
            (hr_b1, hi_b1, tr_b1, ti_b1, rr_b1, ri_b1)]
    sems = [sem0, sem1]

    def fire(ch):
        hr_b, hi_b, tr_b, ti_b, rr_b, ri_b = bufs[ch % 2]
        sem = sems[ch % 2]
        hsl = hp_v.at[pl.ds(ch * _C, _C)]
        rsl = rp_v.at[pl.ds(ch * _C, _C)]
        tsl = tp_v.at[pl.ds(ch * _C, _C)]
        return [
            pltpu.async_copy(er_hbm.at[hsl], hr_b, sem),
            pltpu.async_copy(ei_hbm.at[hsl], hi_b, sem),
            pltpu.async_copy(er_hbm.at[tsl], tr_b, sem),
            pltpu.async_copy(ei_hbm.at[tsl], ti_b, sem),
            pltpu.async_copy(rr_hbm.at[rsl], rr_b, sem),
            pltpu.async_copy(ri_hbm.at[rsl], ri_b, sem),
        ]

    pending = fire(0)
    for ch in range(_NCH):
        for cp in pending:
            cp.wait()
        if ch + 1 < _NCH:
            pending = fire(ch + 1)
        hr_b, hi_b, tr_b, ti_b, rr_b, ri_b = bufs[ch % 2]

        def grp_body(g, _, ch=ch, hr_b=hr_b, hi_b=hi_b, tr_b=tr_b,
                     ti_b=ti_b, rr_b=rr_b, ri_b=ri_b):
            rows = g * _L + iota
            hc0 = h_iv[pl.ds(ch * _C + g * _L, _L)]
            rc0 = r_iv[pl.ds(ch * _C + g * _L, _L)]
            tc0 = t_iv[pl.ds(ch * _C + g * _L, _L)]

            def dim_body(d, acc):
                hc = hc0 + d
                rc = rc0 + d
                tc = tc0 + d
                hr = plsc.load_gather(hr_b, [rows, hc])
                hi = plsc.load_gather(hi_b, [rows, hc])
                tr = plsc.load_gather(tr_b, [rows, tc])
                ti = plsc.load_gather(ti_b, [rows, tc])
                rr = plsc.load_gather(rr_b, [rows, rc])
                ri = plsc.load_gather(ri_b, [rows, rc])
                return acc + tr * (hr * rr - hi * ri) + ti * (hi * rr + hr * ri)

            acc = lax.fori_loop(0, _D, dim_body, jnp.zeros((_L,), jnp.float32))
            out_v[pl.ds(ch * _C + g * _L, _L)] = acc
            return 0

        lax.fori_loop(0, _C // _L, grp_body, 0)

    pltpu.sync_copy(out_v, out_hbm.at[pl.ds(base, _EPW)])


@jax.jit
def kernel(head, relation, tail, ent_real, ent_imag, rel_real, rel_imag):
    er2, ei2 = _pair_rows(ent_real.T, ent_imag.T, _EBLK_ENT)
    rr2, ri2 = _pair_rows(rel_real.T, rel_imag.T, _EBLK_REL)
    mesh = plsc.VectorSubcoreMesh(core_axis_name="c", subcore_axis_name="s")
    run = pl.kernel(
        _complex_body,
        out_type=jax.ShapeDtypeStruct((_B,), jnp.float32),
        mesh=mesh,
        scratch_types=[
            pltpu.VMEM((_EPW,), jnp.int32),
            pltpu.VMEM((_EPW,), jnp.int32),
            pltpu.VMEM((_EPW,), jnp.int32),
            pltpu.VMEM((_EPW,), jnp.int32),
            pltpu.VMEM((_EPW,), jnp.int32),
            pltpu.VMEM((_EPW,), jnp.int32),
            pltpu.VMEM((_EPW,), jnp.float32),
            pltpu.VMEM((_C, 2 * _D), jnp.float32),
            pltpu.VMEM((_C, 2 * _D), jnp.float32),
            pltpu.VMEM((_C, 2 * _D), jnp.float32),
            pltpu.VMEM((_C, 2 * _D), jnp.float32),
            pltpu.VMEM((_C, 2 * _D), jnp.float32),
            pltpu.VMEM((_C, 2 * _D), jnp.float32),
            pltpu.VMEM((_C, 2 * _D), jnp.float32),
            pltpu.VMEM((_C, 2 * _D), jnp.float32),
            pltpu.VMEM((_C, 2 * _D), jnp.float32),
            pltpu.VMEM((_C, 2 * _D), jnp.float32),
            pltpu.VMEM((_C, 2 * _D), jnp.float32),
            pltpu.VMEM((_C, 2 * _D), jnp.float32),
            pltpu.SemaphoreType.DMA,
            pltpu.SemaphoreType.DMA,
        ],
        compiler_params=pltpu.CompilerParams(
            needs_layout_passes=False, use_tc_tiling_on_sc=True),
    )
    return run(head, relation, tail, er2, ei2, rr2, ri2)
